# baseline (speedup 1.0000x reference)
import functools

import jax
import jax.numpy as jnp
from jax import lax
from jax.experimental import pallas as pl
from jax.experimental.pallas import tpu as pltpu
from jax.experimental.pallas import tpu_sc as plsc

N = 10000
E = 320000
D = 128
EPS = 1e-5

NC = 2
NS = 16
NW = NC * NS
EB = 128
KB = 80
GB = 16
E_PAD = NW * KB * EB
N_PAD = 10240
ROWS_PER_TILE = N_PAD // NS
BLK = 1000
NBLK = N // BLK


def _sc_body(feat_hbm, src_hbm, dst_hbm,
             agg_out, deg_out,
             src_v, dst_v, buf0, buf1, deg_loc, sem0, sem1, sem2, sem3,
             agg_sh):
    cid = lax.axis_index("c")
    sid = lax.axis_index("s")
    tid = cid * NS + sid

    zero16 = jnp.zeros((16,), jnp.float32)

    def zrow(r, carry):
        for c in range(D // 16):
            buf0[r, pl.ds(c * 16, 16)] = zero16
        return carry

    lax.fori_loop(0, EB, zrow, 0)

    def zdeg(i, carry):
        deg_loc[pl.ds(i * 16, 16)] = zero16
        return carry

    lax.fori_loop(0, N_PAD // 16, zdeg, 0)

    r0 = sid * ROWS_PER_TILE
    for k in range(ROWS_PER_TILE // EB):
        pltpu.sync_copy(buf0, agg_sh.at[pl.ds(r0 + k * EB, EB)])

    plsc.subcore_barrier()

    def hist(j):
        for c in range(EB // 16):
            idx16 = dst_v[j, pl.ds(c * 16, 16)]
            cnt, last = plsc.scan_count(idx16)
            plsc.addupdate_scatter(
                deg_loc, [idx16], cnt.astype(jnp.float32), mask=last)

    def outer(g, carry):
        rows = tid * KB + g * GB + lax.iota(jnp.int32, GB)
        pltpu.sync_copy(src_hbm.at[rows], src_v)
        pltpu.sync_copy(dst_hbm.at[rows], dst_v)

        pltpu.async_copy(feat_hbm.at[src_v.at[0]], buf0, sem0)

        def pair(p, c2):
            j0 = 2 * p

            @pl.when(p > 0)
            def _():
                pltpu.make_async_copy(
                    buf1, agg_sh.at[dst_v.at[j0 - 1]], sem3).wait()

            pltpu.async_copy(feat_hbm.at[src_v.at[j0 + 1]], buf1, sem1)
            hist(j0)
            pltpu.make_async_copy(
                feat_hbm.at[src_v.at[j0]], buf0, sem0).wait()
            pltpu.async_copy(buf0, agg_sh.at[dst_v.at[j0]], sem2, add=True)
            hist(j0 + 1)
            pltpu.make_async_copy(
                feat_hbm.at[src_v.at[j0 + 1]], buf1, sem1).wait()
            pltpu.make_async_copy(
                buf0, agg_sh.at[dst_v.at[j0]], sem2).wait()

            @pl.when(p + 1 < GB // 2)
            def _():
                pltpu.async_copy(feat_hbm.at[src_v.at[j0 + 2]], buf0, sem0)

            pltpu.async_copy(
                buf1, agg_sh.at[dst_v.at[j0 + 1]], sem3, add=True)
            return c2

        lax.fori_loop(0, GB // 2, pair, 0)
        pltpu.make_async_copy(
            buf1, agg_sh.at[dst_v.at[GB - 1]], sem3).wait()
        return carry

    lax.fori_loop(0, KB // GB, outer, 0)

    plsc.subcore_barrier()

    last = N - (NS - 1) * ROWS_PER_TILE

    @pl.when(sid < NS - 1)
    def _():
        pltpu.sync_copy(agg_sh.at[pl.ds(r0, ROWS_PER_TILE)],
                        agg_out.at[cid, pl.ds(r0, ROWS_PER_TILE)])

    @pl.when(sid == NS - 1)
    def _():
        pltpu.sync_copy(agg_sh.at[pl.ds(r0, last)],
                        agg_out.at[cid, pl.ds(r0, last)])

    pltpu.sync_copy(deg_loc, deg_out.at[tid])


_sc_scatter = pl.kernel(
    _sc_body,
    out_type=(
        jax.ShapeDtypeStruct((NC, N, D), jnp.float32),
        jax.ShapeDtypeStruct((NW, N_PAD), jnp.float32),
    ),
    mesh=plsc.VectorSubcoreMesh(core_axis_name="c", subcore_axis_name="s"),
    compiler_params=pltpu.CompilerParams(needs_layout_passes=False),
    scratch_types=[
        pltpu.VMEM((GB, EB), jnp.int32),
        pltpu.VMEM((GB, EB), jnp.int32),
        pltpu.VMEM((EB, D), jnp.float32),
        pltpu.VMEM((EB, D), jnp.float32),
        pltpu.VMEM((N_PAD,), jnp.float32),
        pltpu.SemaphoreType.DMA,
        pltpu.SemaphoreType.DMA,
        pltpu.SemaphoreType.DMA,
        pltpu.SemaphoreType.DMA,
        pltpu.VMEM_SHARED((N_PAD, D), jnp.float32),
    ],
)


_INV_SQRT2 = 0.7071067811865476


def _tc_body(f_ref, agg_ref, deg_ref, g_ref, bt_ref, ws_ref, wn_ref, b_ref,
             o_ref, acc_ref, st_ref):
    ph = pl.program_id(0)
    i = pl.program_id(1)

    @pl.when(ph == 0)
    def _():
        @pl.when(i == 0)
        def _():
            acc_ref[...] = jnp.zeros_like(acc_ref)

        x = f_ref[...]
        acc_ref[0:1, :] += jnp.sum(x, axis=0, keepdims=True)
        acc_ref[1:2, :] += jnp.sum(x * x, axis=0, keepdims=True)

        @pl.when(i == NBLK - 1)
        def _():
            mean = acc_ref[0:1, :] * (1.0 / N)
            var = acc_ref[1:2, :] * (1.0 / N) - mean * mean
            s = g_ref[...] * lax.rsqrt(var + EPS)
            st_ref[0:1, :] = s
            st_ref[1:2, :] = bt_ref[...] - mean * s

    @pl.when(ph == 1)
    def _():
        s = st_ref[0:1, :]
        t = st_ref[1:2, :]
        h = f_ref[...] * s + t
        a = agg_ref[0] + agg_ref[1]
        d = jnp.sum(deg_ref[...], axis=1, keepdims=True)
        hn = (a / jnp.maximum(d, 1.0)) * s + jnp.where(d > 0.0, t, 0.0)
        conv = (
            jnp.dot(h, ws_ref[...], preferred_element_type=jnp.float32)
            + jnp.dot(hn, wn_ref[...], preferred_element_type=jnp.float32)
            + b_ref[...]
            + h
        )
        o_ref[...] = 0.5 * conv * (1.0 + lax.erf(conv * _INV_SQRT2))


def _tc_call(features, agg2, degT, gamma, beta, W_self, W_neigh, b):
    return pl.pallas_call(
        _tc_body,
        grid=(2, NBLK),
        in_specs=[
            pl.BlockSpec((BLK, D), lambda ph, i: (i, 0)),
            pl.BlockSpec((NC, BLK, D), lambda ph, i: (0, ph * i, 0)),
            pl.BlockSpec((BLK, NW), lambda ph, i: (ph * i, 0)),
            pl.BlockSpec((1, D), lambda ph, i: (0, 0)),
            pl.BlockSpec((1, D), lambda ph, i: (0, 0)),
            pl.BlockSpec((D, D), lambda ph, i: (0, 0)),
            pl.BlockSpec((D, D), lambda ph, i: (0, 0)),
            pl.BlockSpec((1, D), lambda ph, i: (0, 0)),
        ],
        out_specs=pl.BlockSpec((BLK, D), lambda ph, i: (i, 0)),
        out_shape=jax.ShapeDtypeStruct((N, D), jnp.float32),
        scratch_shapes=[pltpu.VMEM((2, D), jnp.float32),
                        pltpu.VMEM((2, D), jnp.float32)],
    )(features, agg2, degT, gamma, beta, W_self, W_neigh, b)


def kernel(features, edge_index, W_self, W_neigh, b, gamma, beta):
    src = edge_index[0]
    dst = edge_index[1]
    pad = E_PAD - E
    pad_dst = N + (jnp.arange(pad, dtype=jnp.int32) % (N_PAD - N))
    pad_src = jnp.arange(pad, dtype=jnp.int32) * 37 % N
    src3 = jnp.concatenate([src, pad_src]).reshape(NW * KB, EB)
    dst3 = jnp.concatenate([dst, pad_dst]).reshape(NW * KB, EB)

    agg2, deg2 = _sc_scatter(features, src3, dst3)

    return _tc_call(features, agg2, deg2.T, gamma.reshape(1, D),
                    beta.reshape(1, D), W_self, W_neigh, b.reshape(1, D))


# speedup vs baseline: 1.0260x; 1.0260x over previous
import functools

import jax
import jax.numpy as jnp
from jax import lax
from jax.experimental import pallas as pl
from jax.experimental.pallas import tpu as pltpu
from jax.experimental.pallas import tpu_sc as plsc

N = 10000
E = 320000
D = 128
EPS = 1e-5

NC = 2
NS = 16
NW = NC * NS
EB = 128
KB = 80
GB = 16
E_PAD = NW * KB * EB
N_PAD = 10240
ROWS_PER_TILE = N_PAD // NS
BLK = 1000
NBLK = N // BLK


def _sc_body(feat_hbm, src_hbm, dst_hbm,
             agg_out, deg_out,
             src_v, dst_v, buf0, buf1, deg_loc, sem0, sem1, agg_sh):
    cid = lax.axis_index("c")
    sid = lax.axis_index("s")
    tid = cid * NS + sid

    zero16 = jnp.zeros((16,), jnp.float32)

    def zrow(r, carry):
        for c in range(D // 16):
            buf0[r, pl.ds(c * 16, 16)] = zero16
        return carry

    lax.fori_loop(0, EB, zrow, 0)

    def zdeg(i, carry):
        deg_loc[pl.ds(i * 16, 16)] = zero16
        return carry

    lax.fori_loop(0, N_PAD // 16, zdeg, 0)

    r0 = sid * ROWS_PER_TILE
    for k in range(ROWS_PER_TILE // EB):
        pltpu.sync_copy(buf0, agg_sh.at[pl.ds(r0 + k * EB, EB)])

    plsc.subcore_barrier()

    def hist(j):
        for c in range(EB // 16):
            idx16 = dst_v[j, pl.ds(c * 16, 16)]
            cnt, last = plsc.scan_count(idx16)
            plsc.addupdate_scatter(
                deg_loc, [idx16], cnt.astype(jnp.float32), mask=last)

    def outer(g, carry):
        rows = tid * KB + g * GB + lax.iota(jnp.int32, GB)
        pltpu.sync_copy(src_hbm.at[rows], src_v)
        pltpu.sync_copy(dst_hbm.at[rows], dst_v)

        pltpu.async_copy(feat_hbm.at[src_v.at[0]], buf0, sem0)

        def pair(p, c2):
            j0 = 2 * p
            pltpu.async_copy(feat_hbm.at[src_v.at[j0 + 1]], buf1, sem1)
            hist(j0)
            pltpu.make_async_copy(
                feat_hbm.at[src_v.at[j0]], buf0, sem0).wait()
            pltpu.sync_copy(buf0, agg_sh.at[dst_v.at[j0]], add=True)

            @pl.when(p + 1 < GB // 2)
            def _():
                pltpu.async_copy(feat_hbm.at[src_v.at[j0 + 2]], buf0, sem0)

            hist(j0 + 1)
            pltpu.make_async_copy(
                feat_hbm.at[src_v.at[j0 + 1]], buf1, sem1).wait()
            pltpu.sync_copy(buf1, agg_sh.at[dst_v.at[j0 + 1]], add=True)
            return c2

        lax.fori_loop(0, GB // 2, pair, 0)
        return carry

    lax.fori_loop(0, KB // GB, outer, 0)

    plsc.subcore_barrier()

    last = N - (NS - 1) * ROWS_PER_TILE

    @pl.when(sid < NS - 1)
    def _():
        pltpu.sync_copy(agg_sh.at[pl.ds(r0, ROWS_PER_TILE)],
                        agg_out.at[cid, pl.ds(r0, ROWS_PER_TILE)])

    @pl.when(sid == NS - 1)
    def _():
        pltpu.sync_copy(agg_sh.at[pl.ds(r0, last)],
                        agg_out.at[cid, pl.ds(r0, last)])

    pltpu.sync_copy(deg_loc, deg_out.at[tid])


_sc_scatter = pl.kernel(
    _sc_body,
    out_type=(
        jax.ShapeDtypeStruct((NC, N, D), jnp.float32),
        jax.ShapeDtypeStruct((NW, N_PAD), jnp.float32),
    ),
    mesh=plsc.VectorSubcoreMesh(core_axis_name="c", subcore_axis_name="s"),
    compiler_params=pltpu.CompilerParams(needs_layout_passes=False),
    scratch_types=[
        pltpu.VMEM((GB, EB), jnp.int32),
        pltpu.VMEM((GB, EB), jnp.int32),
        pltpu.VMEM((EB, D), jnp.float32),
        pltpu.VMEM((EB, D), jnp.float32),
        pltpu.VMEM((N_PAD,), jnp.float32),
        pltpu.SemaphoreType.DMA,
        pltpu.SemaphoreType.DMA,
        pltpu.VMEM_SHARED((N_PAD, D), jnp.float32),
    ],
)


_INV_SQRT2 = 0.7071067811865476


def _tc_body(f_ref, agg_ref, deg_ref, g_ref, bt_ref, ws_ref, wn_ref, b_ref,
             o_ref, acc_ref, st_ref):
    ph = pl.program_id(0)
    i = pl.program_id(1)

    @pl.when(ph == 0)
    def _():
        @pl.when(i == 0)
        def _():
            acc_ref[...] = jnp.zeros_like(acc_ref)

        x = f_ref[...]
        acc_ref[0:1, :] += jnp.sum(x, axis=0, keepdims=True)
        acc_ref[1:2, :] += jnp.sum(x * x, axis=0, keepdims=True)

        @pl.when(i == NBLK - 1)
        def _():
            mean = acc_ref[0:1, :] * (1.0 / N)
            var = acc_ref[1:2, :] * (1.0 / N) - mean * mean
            s = g_ref[...] * lax.rsqrt(var + EPS)
            st_ref[0:1, :] = s
            st_ref[1:2, :] = bt_ref[...] - mean * s

    @pl.when(ph == 1)
    def _():
        s = st_ref[0:1, :]
        t = st_ref[1:2, :]
        h = f_ref[...] * s + t
        a = agg_ref[0] + agg_ref[1]
        d = jnp.sum(deg_ref[...], axis=1, keepdims=True)
        hn = (a / jnp.maximum(d, 1.0)) * s + jnp.where(d > 0.0, t, 0.0)
        conv = (
            jnp.dot(h, ws_ref[...], preferred_element_type=jnp.float32)
            + jnp.dot(hn, wn_ref[...], preferred_element_type=jnp.float32)
            + b_ref[...]
            + h
        )
        o_ref[...] = 0.5 * conv * (1.0 + lax.erf(conv * _INV_SQRT2))


def _tc_call(features, agg2, degT, gamma, beta, W_self, W_neigh, b):
    return pl.pallas_call(
        _tc_body,
        grid=(2, NBLK),
        in_specs=[
            pl.BlockSpec((BLK, D), lambda ph, i: (i, 0)),
            pl.BlockSpec((NC, BLK, D), lambda ph, i: (0, ph * i, 0)),
            pl.BlockSpec((BLK, NW), lambda ph, i: (ph * i, 0)),
            pl.BlockSpec((1, D), lambda ph, i: (0, 0)),
            pl.BlockSpec((1, D), lambda ph, i: (0, 0)),
            pl.BlockSpec((D, D), lambda ph, i: (0, 0)),
            pl.BlockSpec((D, D), lambda ph, i: (0, 0)),
            pl.BlockSpec((1, D), lambda ph, i: (0, 0)),
        ],
        out_specs=pl.BlockSpec((BLK, D), lambda ph, i: (i, 0)),
        out_shape=jax.ShapeDtypeStruct((N, D), jnp.float32),
        scratch_shapes=[pltpu.VMEM((2, D), jnp.float32),
                        pltpu.VMEM((2, D), jnp.float32)],
    )(features, agg2, degT, gamma, beta, W_self, W_neigh, b)


def kernel(features, edge_index, W_self, W_neigh, b, gamma, beta):
    src = edge_index[0]
    dst = edge_index[1]
    pad = E_PAD - E
    pad_dst = N + (jnp.arange(pad, dtype=jnp.int32) % (N_PAD - N))
    pad_src = jnp.arange(pad, dtype=jnp.int32) * 37 % N
    src3 = jnp.concatenate([src, pad_src]).reshape(NW * KB, EB)
    dst3 = jnp.concatenate([dst, pad_dst]).reshape(NW * KB, EB)

    agg2, deg2 = _sc_scatter(features, src3, dst3)

    return _tc_call(features, agg2, deg2.T, gamma.reshape(1, D),
                    beta.reshape(1, D), W_self, W_neigh, b.reshape(1, D))
